# baseline (speedup 1.0000x reference)
import functools

import jax
import jax.numpy as jnp
from jax import lax
from jax.experimental import pallas as pl
from jax.experimental.pallas import tpu as pltpu
from jax.experimental.pallas import tpu_sc as plsc

NC = 2
NS = 16
NW = NC * NS

B = 4096
S = 200
D = 64
CHUNK_OFF = (0, 128)
CHUNK_LEN = (128, 72)
ROWS_PER_W = B // NW
RING = 4
NITER = ROWS_PER_W // RING


def _make_kernel():
  mesh = plsc.VectorSubcoreMesh(core_axis_name="c", subcore_axis_name="s")

  @functools.partial(
      pl.kernel,
      out_type=jax.ShapeDtypeStruct((B, S, D), jnp.float32),
      mesh=mesh,
      compiler_params=pltpu.CompilerParams(use_tc_tiling_on_sc=False),
      scratch_types=[
          pltpu.VMEM((ROWS_PER_W, S), jnp.int32),
          [pltpu.VMEM((S, D), jnp.float32)] * RING,
          [pltpu.SemaphoreType.DMA] * RING,
          [pltpu.SemaphoreType.DMA] * RING,
      ],
  )
  def k(idx_hbm, table_hbm, out_hbm, idx_v, bufs, gsems, wsems):
    wid = lax.axis_index("s") * NC + lax.axis_index("c")
    rbase = wid * ROWS_PER_W
    pltpu.sync_copy(idx_hbm.at[pl.ds(rbase, ROWS_PER_W)], idx_v)

    def step(i, _):
      for b in range(RING):
        r = RING * i + b
        @pl.when(i > 0)
        def _():
          pltpu.make_async_copy(bufs[b], out_hbm.at[rbase], wsems[b]).wait()
        for c in range(2):
          pltpu.async_copy(
              table_hbm.at[idx_v.at[r, pl.ds(CHUNK_OFF[c], CHUNK_LEN[c])]],
              bufs[b].at[pl.ds(CHUNK_OFF[c], CHUNK_LEN[c])], gsems[b])
      for b in range(RING):
        r = RING * i + b
        for c in range(2):
          pltpu.make_async_copy(
              table_hbm.at[idx_v.at[r, pl.ds(CHUNK_OFF[c], CHUNK_LEN[c])]],
              bufs[b].at[pl.ds(CHUNK_OFF[c], CHUNK_LEN[c])], gsems[b]).wait()
        pltpu.async_copy(bufs[b], out_hbm.at[rbase + r], wsems[b])
      return 0

    lax.fori_loop(0, NITER, step, 0)

    for b in range(RING):
      pltpu.make_async_copy(bufs[b], out_hbm.at[rbase], wsems[b]).wait()

  return k


_gather_kernel = _make_kernel()


@jax.jit
def kernel(x, embedding):
  return _gather_kernel(x.astype(jnp.int32), embedding)


# speedup vs baseline: 1.3288x; 1.3288x over previous
import functools

import jax
import jax.numpy as jnp
from jax import lax
from jax.experimental import pallas as pl
from jax.experimental.pallas import tpu as pltpu
from jax.experimental.pallas import tpu_sc as plsc

NC = 2
NS = 16
NW = NC * NS

B = 4096
S = 200
D = 64
CHUNK_OFF = (0, 128)
CHUNK_LEN = (128, 72)
ROWS_PER_W = B // NW
RING = 4
NITER = ROWS_PER_W // RING


def _make_kernel():
  mesh = plsc.VectorSubcoreMesh(core_axis_name="c", subcore_axis_name="s")

  @functools.partial(
      pl.kernel,
      out_type=jax.ShapeDtypeStruct((B, S, 2 * D), jnp.float32),
      mesh=mesh,
      compiler_params=pltpu.CompilerParams(use_tc_tiling_on_sc=False),
      scratch_types=[
          pltpu.VMEM((ROWS_PER_W, S), jnp.int32),
          [pltpu.VMEM((S, D), jnp.float32)] * RING,
          [pltpu.SemaphoreType.DMA] * RING,
          [pltpu.SemaphoreType.DMA] * RING,
      ],
  )
  def k(idx_hbm, table_hbm, out_hbm, idx_v, bufs, gsems, wsems):
    wid = lax.axis_index("s") * NC + lax.axis_index("c")
    rbase = wid * ROWS_PER_W
    pltpu.sync_copy(idx_hbm.at[pl.ds(rbase, ROWS_PER_W)], idx_v)

    def step(i, _):
      for b in range(RING):
        r = RING * i + b
        @pl.when(i > 0)
        def _():
          pltpu.make_async_copy(
              bufs[b],
              out_hbm.at[rbase, pl.ds(0, S), pl.ds(0, D)], wsems[b]).wait()
        for c in range(2):
          pltpu.async_copy(
              table_hbm.at[idx_v.at[r, pl.ds(CHUNK_OFF[c], CHUNK_LEN[c])]],
              bufs[b].at[pl.ds(CHUNK_OFF[c], CHUNK_LEN[c])], gsems[b])
      for b in range(RING):
        r = RING * i + b
        for c in range(2):
          pltpu.make_async_copy(
              table_hbm.at[idx_v.at[r, pl.ds(CHUNK_OFF[c], CHUNK_LEN[c])]],
              bufs[b].at[pl.ds(CHUNK_OFF[c], CHUNK_LEN[c])], gsems[b]).wait()
        pltpu.async_copy(
            bufs[b],
            out_hbm.at[rbase + r, pl.ds(0, S), pl.ds(0, D)], wsems[b])
      return 0

    lax.fori_loop(0, NITER, step, 0)

    for b in range(RING):
      pltpu.make_async_copy(
          bufs[b],
          out_hbm.at[rbase, pl.ds(0, S), pl.ds(0, D)], wsems[b]).wait()

  return k


_gather_kernel = _make_kernel()


@jax.jit
def kernel(x, embedding):
  out = _gather_kernel(x.astype(jnp.int32), embedding)
  return out[:, :, :D]
